# baseline (speedup 1.0000x reference)
import functools

import jax
import jax.numpy as jnp
from jax import lax
from jax.experimental import pallas as pl
from jax.experimental.pallas import tpu as pltpu
from jax.experimental.pallas import tpu_sc as plsc

EMBED = 1024
NUM_HOPS = 2
GAMMA = 0.8
B, L, V, MT, MC = 4, 128, 32000, 1024, 512

MTB = 128
NLG = L // 16
CH = 3200
VB = 3200
CPT_W = MC + 8


def _tc_dense_body(trep_ref, w_ref, hid_ref, tl_ref, wg_ref, bg_ref,
                   tp_ref, tpt_ref, gate_ref, tmg_ref):
    tr = trep_ref[0]
    w = w_ref[...]
    emb = lax.dot_general(tr, w, (((1,), (1,)), ((), ())),
                          preferred_element_type=jnp.float32)
    hid = hid_ref[0]
    logits = lax.dot_general(hid, emb, (((1,), (1,)), ((), ())),
                             preferred_element_type=jnp.float32)
    tl = tl_ref[0, 0, :]
    tmask = (tl != -1).astype(jnp.float32)
    tp = jax.nn.sigmoid(logits) * tmask[None, :]
    tp_ref[0] = tp
    tpt_ref[0] = jnp.transpose(tp.reshape(NLG, 16, MTB),
                               (0, 2, 1)).reshape(NLG, MTB * 16)
    tmg_ref[0, 0] = tmask * GAMMA
    wg = wg_ref[0]
    gsum = jnp.sum(hid * wg[None, :], axis=-1) + bg_ref[0, 0]
    gate_ref[0, 0] = jax.nn.sigmoid(gsum)


def _tc_dense(triple_repr, w_triple, hidden, tlabels, w_gate, b_gate):
    tl3 = tlabels.reshape(B, 1, MT)
    bg2 = b_gate.reshape(1, 1)
    grid = (B, MT // MTB)
    return pl.pallas_call(
        _tc_dense_body,
        grid=grid,
        in_specs=[
            pl.BlockSpec((1, MTB, 3 * EMBED), lambda b, m: (b, m, 0)),
            pl.BlockSpec((EMBED, 3 * EMBED), lambda b, m: (0, 0)),
            pl.BlockSpec((1, L, EMBED), lambda b, m: (b, 0, 0)),
            pl.BlockSpec((1, 1, MTB), lambda b, m: (b, 0, m)),
            pl.BlockSpec((1, EMBED), lambda b, m: (0, 0)),
            pl.BlockSpec((1, 1), lambda b, m: (0, 0)),
        ],
        out_specs=[
            pl.BlockSpec((1, L, MTB), lambda b, m: (b, 0, m)),
            pl.BlockSpec((1, NLG, MTB * 16), lambda b, m: (b, 0, m)),
            pl.BlockSpec((1, 1, L), lambda b, m: (b, 0, 0)),
            pl.BlockSpec((1, 1, MTB), lambda b, m: (b, 0, m)),
        ],
        out_shape=[
            jax.ShapeDtypeStruct((B, L, MT), jnp.float32),
            jax.ShapeDtypeStruct((B, NLG, MT * 16), jnp.float32),
            jax.ShapeDtypeStruct((B, 1, L), jnp.float32),
            jax.ShapeDtypeStruct((B, 1, MT), jnp.float32),
        ],
        compiler_params=pltpu.CompilerParams(
            dimension_semantics=("parallel", "arbitrary")),
    )(triple_repr, w_triple, hidden, tl3, w_gate, bg2)


def _tc_mapm_body(vm_ref, mm_ref, out_ref):
    out_ref[0, 0] = jnp.where(mm_ref[0, 0] != 0, vm_ref[0, 0], MC)


def _tc_mapm(vocab_map, map_mask):
    vm3 = vocab_map.reshape(B, 1, V)
    mm3 = map_mask.reshape(B, 1, V)
    out = pl.pallas_call(
        _tc_mapm_body,
        grid=(B,),
        in_specs=[pl.BlockSpec((1, 1, V), lambda b: (b, 0, 0)),
                  pl.BlockSpec((1, 1, V), lambda b: (b, 0, 0))],
        out_specs=pl.BlockSpec((1, 1, V), lambda b: (b, 0, 0)),
        out_shape=jax.ShapeDtypeStruct((B, 1, V), jnp.int32),
    )(vm3, mm3)
    return out.reshape(B, V)


def _tc_isc_body(p_ref, lm_ref, isc_ref, pv_ref, pi_ref, lv_ref, li_ref):
    m = pl.program_id(1)

    def upd(x_ref, bv_ref, bi_ref):
        x = x_ref[0]
        loc = jnp.max(x, axis=-1)
        idx = jnp.argmax(x, axis=-1).astype(jnp.int32) + m * VB
        better = loc > bv_ref[0, 0]
        bi_ref[0, 0] = jnp.where(better, idx, bi_ref[0, 0])
        bv_ref[0, 0] = jnp.where(better, loc, bv_ref[0, 0])

    @pl.when(m == 0)
    def _():
        pv_ref[0, 0] = jnp.full((L,), -1.0, jnp.float32)
        lv_ref[0, 0] = jnp.full((L,), -1.0, jnp.float32)
        pi_ref[0, 0] = jnp.zeros((L,), jnp.int32)
        li_ref[0, 0] = jnp.zeros((L,), jnp.int32)

    upd(p_ref, pv_ref, pi_ref)
    upd(lm_ref, lv_ref, li_ref)

    @pl.when(m == V // VB - 1)
    def _():
        isc_ref[0, 0] = (pi_ref[0, 0] != li_ref[0, 0]).astype(jnp.int32)


def _tc_isc(probs, lm_probs):
    outs = pl.pallas_call(
        _tc_isc_body,
        grid=(B, V // VB),
        in_specs=[pl.BlockSpec((1, L, VB), lambda b, m: (b, 0, m)),
                  pl.BlockSpec((1, L, VB), lambda b, m: (b, 0, m))],
        out_specs=[pl.BlockSpec((1, 1, L), lambda b, m: (b, 0, 0))] * 5,
        out_shape=[jax.ShapeDtypeStruct((B, 1, L), jnp.int32),
                   jax.ShapeDtypeStruct((B, 1, L), jnp.float32),
                   jax.ShapeDtypeStruct((B, 1, L), jnp.int32),
                   jax.ShapeDtypeStruct((B, 1, L), jnp.float32),
                   jax.ShapeDtypeStruct((B, 1, L), jnp.int32)],
        compiler_params=pltpu.CompilerParams(
            dimension_semantics=("parallel", "arbitrary")),
    )(probs, lm_probs)
    return outs[0].reshape(B, L)


def _iota16():
    return lax.iota(jnp.int32, 16)


def _full16i(x):
    return jnp.full((16,), x, jnp.int32)


def _sc_body(tpt_hbm, gate_hbm, lm_hbm, mapm_hbm, head_hbm, tail_hbm,
             tmg_hbm, dist_hbm, clab_hbm,
             probs_hbm, cpv_hbm,
             tpv, s0v, d0v, cmv, s1v, s2v, headv, tailv, tmgv, distv, clabv,
             cptf, gatev, mapmv, lmst, pst, cst):
    wid = lax.axis_index("s") * 2 + lax.axis_index("c")
    b = wid // NLG
    lg = wid % NLG
    it = _iota16()
    zv = jnp.zeros((16,), jnp.float32)

    pltpu.sync_copy(tpt_hbm.at[b, lg], tpv)
    pltpu.sync_copy(gate_hbm.at[b, 0], gatev)
    pltpu.sync_copy(head_hbm.at[b], headv)
    pltpu.sync_copy(tail_hbm.at[b], tailv)
    pltpu.sync_copy(tmg_hbm.at[b, 0], tmgv)
    pltpu.sync_copy(dist_hbm.at[b], distv)
    pltpu.sync_copy(clab_hbm.at[b], clabv)
    pltpu.sync_copy(mapm_hbm.at[b], mapmv)

    def init_grp(g, _):
        sl = pl.ds(g * 16, 16)
        dv = distv[sl]
        cl = clabv[sl]
        d016 = jnp.where(dv == 0, 1.0, 0.0)
        cm16 = jnp.where(cl != -1, 1.0, 0.0)
        d0v[sl] = d016
        cmv[sl] = cm16
        s0v[sl] = d016 * cm16
        return 0

    lax.fori_loop(0, MC // 16, init_grp, 0)

    def zero_c(c, _):
        sl = pl.ds(c * 16, 16)
        s1v[sl] = zv
        s2v[sl] = zv
        return 0

    lax.fori_loop(0, MC, zero_c, 0)

    def do_hop(first, dst):
        def hop_grp(grp, _):
            base = grp * 16
            gsl = pl.ds(base, 16)
            hv = headv[gsl]
            tv = tailv[gsl]
            gv = tmgv[gsl]
            for k in range(16):
                if first:
                    ns = plsc.load_gather(s0v, [_full16i(hv[k])])
                else:
                    ns = s1v[pl.ds(hv[k] * 16, 16)]
                upd = ns * gv[k] + tpv[pl.ds((base + k) * 16, 16)]
                sl = pl.ds(tv[k] * 16, 16)
                dst[sl] = jnp.maximum(dst[sl], upd)
            return 0

        lax.fori_loop(0, MT // 16, hop_grp, 0)

    do_hop(True, s1v)

    def mask_c(c, _):
        sl = pl.ds(c * 16, 16)
        s1v[sl] = s1v[sl] * plsc.load_gather(cmv, [_full16i(c)])
        return 0

    lax.fori_loop(0, MC, mask_c, 0)

    do_hop(False, s2v)

    itW = it * CPT_W

    def _tot_body(c, m):
        sl = pl.ds(c * 16, 16)
        d016 = plsc.load_gather(d0v, [_full16i(c)])
        cm16 = plsc.load_gather(cmv, [_full16i(c)])
        tv = (d016 + s1v[sl]) + s2v[sl] * cm16
        s1v[sl] = tv
        return jnp.maximum(m, tv)

    mx = lax.fori_loop(0, MC, _tot_body, jnp.full((16,), -jnp.inf))

    def _exp_body(c, acc):
        sl = pl.ds(c * 16, 16)
        e = jnp.exp(s1v[sl] - mx)
        s2v[sl] = e
        return acc + e

    ssum = lax.fori_loop(0, MC, _exp_body, zv)

    plsc.store_scatter(cptf, [itW + MC], zv)

    def norm_c(c, _):
        e = s2v[pl.ds(c * 16, 16)] / ssum
        plsc.store_scatter(cptf, [itW + c], e)
        return 0

    lax.fori_loop(0, MC, norm_c, 0)

    gv16 = gatev[pl.ds(lg * 16, 16)]

    for l_loc in range(16):
        g = gv16[l_loc]
        om = 1.0 - g
        l_glob = lg * 16 + l_loc
        row_base = l_loc * CPT_W

        def chunk_body(j, _, g=g, om=om, l_glob=l_glob, row_base=row_base):
            pltpu.sync_copy(lm_hbm.at[b, l_glob, pl.ds(j * CH, CH)], lmst)

            @plsc.parallel_loop(0, CH // 16, unroll=8)
            def _vec_body(i):
                o16 = i * 16
                sl = pl.ds(o16, 16)
                mapv = mapmv[pl.ds(j * CH + o16, 16)]
                cv = plsc.load_gather(cptf, [mapv + row_base])
                pst[sl] = cv * g + lmst[sl] * om
                cst[sl] = cv

            pltpu.sync_copy(pst, probs_hbm.at[b, l_glob, pl.ds(j * CH, CH)])
            pltpu.sync_copy(cst, cpv_hbm.at[b, l_glob, pl.ds(j * CH, CH)])
            return 0

        lax.fori_loop(0, V // CH, chunk_body, 0)


def _sc_call(tpt, gate2, lm_probs, mapm, head_idx, tail_idx, tmg,
             distances, concept_labels):
    mesh = plsc.VectorSubcoreMesh(core_axis_name="c", subcore_axis_name="s")
    fn = pl.kernel(
        _sc_body,
        out_type=[
            jax.ShapeDtypeStruct((B, L, V), jnp.float32),
            jax.ShapeDtypeStruct((B, L, V), jnp.float32),
        ],
        mesh=mesh,
        scratch_types=[
            pltpu.VMEM((MT * 16,), jnp.float32),
            pltpu.VMEM((MC,), jnp.float32),
            pltpu.VMEM((MC,), jnp.float32),
            pltpu.VMEM((MC,), jnp.float32),
            pltpu.VMEM((MC * 16,), jnp.float32),
            pltpu.VMEM((MC * 16,), jnp.float32),
            pltpu.VMEM((MT,), jnp.int32),
            pltpu.VMEM((MT,), jnp.int32),
            pltpu.VMEM((MT,), jnp.float32),
            pltpu.VMEM((MC,), jnp.int32),
            pltpu.VMEM((MC,), jnp.int32),
            pltpu.VMEM((16 * CPT_W,), jnp.float32),
            pltpu.VMEM((L,), jnp.float32),
            pltpu.VMEM((V,), jnp.int32),
            pltpu.VMEM((CH,), jnp.float32),
            pltpu.VMEM((CH,), jnp.float32),
            pltpu.VMEM((CH,), jnp.float32),
        ],
        compiler_params=pltpu.CompilerParams(needs_layout_passes=False),
    )
    return fn(tpt, gate2, lm_probs, mapm, head_idx, tail_idx, tmg,
              distances, concept_labels)


def kernel(lm_hidden_states, lm_probs, triple_repr, W_triple, W_gate, b_gate,
           triple_labels, vocab_map, map_mask, distances, concept_labels,
           head_idx, tail_idx):
    tp, tpt, gate2, tmg = _tc_dense(triple_repr, W_triple, lm_hidden_states,
                                    triple_labels, W_gate, b_gate)
    mapm = _tc_mapm(vocab_map, map_mask)
    probs, cpv = _sc_call(tpt, gate2, lm_probs, mapm, head_idx,
                          tail_idx, tmg, distances, concept_labels)
    isc = _tc_isc(probs, lm_probs)
    return (probs, gate2.reshape(B, L, 1), cpv, tp, isc)


# speedup vs baseline: 1.1309x; 1.1309x over previous
import functools

import jax
import jax.numpy as jnp
from jax import lax
from jax.experimental import pallas as pl
from jax.experimental.pallas import tpu as pltpu
from jax.experimental.pallas import tpu_sc as plsc

EMBED = 1024
NUM_HOPS = 2
GAMMA = 0.8
B, L, V, MT, MC = 4, 128, 32000, 1024, 512

MTB = 128
NLG = L // 16
CH = 6400
VB = 3200
CPT_W = MC + 8


def _tc_dense_body(trep_ref, w_ref, hid_ref, tl_ref, wg_ref, bg_ref,
                   tp_ref, tpt_ref, gate_ref, tmg_ref):
    tr = trep_ref[0]
    w = w_ref[...]
    emb = lax.dot_general(tr, w, (((1,), (1,)), ((), ())),
                          preferred_element_type=jnp.float32)
    hid = hid_ref[0]
    logits = lax.dot_general(hid, emb, (((1,), (1,)), ((), ())),
                             preferred_element_type=jnp.float32)
    tl = tl_ref[0, 0, :]
    tmask = (tl != -1).astype(jnp.float32)
    tp = jax.nn.sigmoid(logits) * tmask[None, :]
    tp_ref[0] = tp
    tpt_ref[0] = jnp.transpose(tp.reshape(NLG, 16, MTB),
                               (0, 2, 1)).reshape(NLG, MTB * 16)
    tmg_ref[0, 0] = tmask * GAMMA
    wg = wg_ref[0]
    gsum = jnp.sum(hid * wg[None, :], axis=-1) + bg_ref[0, 0]
    gate_ref[0, 0] = jax.nn.sigmoid(gsum)


def _tc_dense(triple_repr, w_triple, hidden, tlabels, w_gate, b_gate):
    tl3 = tlabels.reshape(B, 1, MT)
    bg2 = b_gate.reshape(1, 1)
    grid = (B, MT // MTB)
    return pl.pallas_call(
        _tc_dense_body,
        grid=grid,
        in_specs=[
            pl.BlockSpec((1, MTB, 3 * EMBED), lambda b, m: (b, m, 0)),
            pl.BlockSpec((EMBED, 3 * EMBED), lambda b, m: (0, 0)),
            pl.BlockSpec((1, L, EMBED), lambda b, m: (b, 0, 0)),
            pl.BlockSpec((1, 1, MTB), lambda b, m: (b, 0, m)),
            pl.BlockSpec((1, EMBED), lambda b, m: (0, 0)),
            pl.BlockSpec((1, 1), lambda b, m: (0, 0)),
        ],
        out_specs=[
            pl.BlockSpec((1, L, MTB), lambda b, m: (b, 0, m)),
            pl.BlockSpec((1, NLG, MTB * 16), lambda b, m: (b, 0, m)),
            pl.BlockSpec((1, 1, L), lambda b, m: (b, 0, 0)),
            pl.BlockSpec((1, 1, MTB), lambda b, m: (b, 0, m)),
        ],
        out_shape=[
            jax.ShapeDtypeStruct((B, L, MT), jnp.float32),
            jax.ShapeDtypeStruct((B, NLG, MT * 16), jnp.float32),
            jax.ShapeDtypeStruct((B, 1, L), jnp.float32),
            jax.ShapeDtypeStruct((B, 1, MT), jnp.float32),
        ],
        compiler_params=pltpu.CompilerParams(
            dimension_semantics=("parallel", "arbitrary")),
    )(triple_repr, w_triple, hidden, tl3, w_gate, bg2)


def _tc_mapm_body(vm_ref, mm_ref, out_ref):
    out_ref[0, 0] = jnp.where(mm_ref[0, 0] != 0, vm_ref[0, 0], MC)


def _tc_mapm(vocab_map, map_mask):
    vm3 = vocab_map.reshape(B, 1, V)
    mm3 = map_mask.reshape(B, 1, V)
    out = pl.pallas_call(
        _tc_mapm_body,
        grid=(B,),
        in_specs=[pl.BlockSpec((1, 1, V), lambda b: (b, 0, 0)),
                  pl.BlockSpec((1, 1, V), lambda b: (b, 0, 0))],
        out_specs=pl.BlockSpec((1, 1, V), lambda b: (b, 0, 0)),
        out_shape=jax.ShapeDtypeStruct((B, 1, V), jnp.int32),
    )(vm3, mm3)
    return out.reshape(B, V)


def _tc_isc_body(p_ref, lm_ref, isc_ref, pv_ref, pi_ref, lv_ref, li_ref):
    m = pl.program_id(1)

    def upd(x_ref, bv_ref, bi_ref):
        x = x_ref[0]
        loc = jnp.max(x, axis=-1)
        idx = jnp.argmax(x, axis=-1).astype(jnp.int32) + m * VB
        better = loc > bv_ref[0, 0]
        bi_ref[0, 0] = jnp.where(better, idx, bi_ref[0, 0])
        bv_ref[0, 0] = jnp.where(better, loc, bv_ref[0, 0])

    @pl.when(m == 0)
    def _():
        pv_ref[0, 0] = jnp.full((L,), -1.0, jnp.float32)
        lv_ref[0, 0] = jnp.full((L,), -1.0, jnp.float32)
        pi_ref[0, 0] = jnp.zeros((L,), jnp.int32)
        li_ref[0, 0] = jnp.zeros((L,), jnp.int32)

    upd(p_ref, pv_ref, pi_ref)
    upd(lm_ref, lv_ref, li_ref)

    @pl.when(m == V // VB - 1)
    def _():
        isc_ref[0, 0] = (pi_ref[0, 0] != li_ref[0, 0]).astype(jnp.int32)


def _tc_isc(probs, lm_probs):
    outs = pl.pallas_call(
        _tc_isc_body,
        grid=(B, V // VB),
        in_specs=[pl.BlockSpec((1, L, VB), lambda b, m: (b, 0, m)),
                  pl.BlockSpec((1, L, VB), lambda b, m: (b, 0, m))],
        out_specs=[pl.BlockSpec((1, 1, L), lambda b, m: (b, 0, 0))] * 5,
        out_shape=[jax.ShapeDtypeStruct((B, 1, L), jnp.int32),
                   jax.ShapeDtypeStruct((B, 1, L), jnp.float32),
                   jax.ShapeDtypeStruct((B, 1, L), jnp.int32),
                   jax.ShapeDtypeStruct((B, 1, L), jnp.float32),
                   jax.ShapeDtypeStruct((B, 1, L), jnp.int32)],
        compiler_params=pltpu.CompilerParams(
            dimension_semantics=("parallel", "arbitrary")),
    )(probs, lm_probs)
    return outs[0].reshape(B, L)


def _iota16():
    return lax.iota(jnp.int32, 16)


def _full16i(x):
    return jnp.full((16,), x, jnp.int32)


def _sc_body(tpt_hbm, gate_hbm, lm_hbm, mapm_hbm, head_hbm, tail_hbm,
             tmg_hbm, dist_hbm, clab_hbm,
             probs_hbm, cpv_hbm,
             tpv, s0v, d0v, cmv, s1v, s2v, headv, tailv, tmgv, distv, clabv,
             cptf, gatev, mapmv, lmst, pst, cst):
    wid = lax.axis_index("s") * 2 + lax.axis_index("c")
    b = wid // NLG
    lg = wid % NLG
    it = _iota16()
    zv = jnp.zeros((16,), jnp.float32)

    pltpu.sync_copy(tpt_hbm.at[b, lg], tpv)
    pltpu.sync_copy(gate_hbm.at[b, 0], gatev)
    pltpu.sync_copy(head_hbm.at[b], headv)
    pltpu.sync_copy(tail_hbm.at[b], tailv)
    pltpu.sync_copy(tmg_hbm.at[b, 0], tmgv)
    pltpu.sync_copy(dist_hbm.at[b], distv)
    pltpu.sync_copy(clab_hbm.at[b], clabv)
    pltpu.sync_copy(mapm_hbm.at[b], mapmv)

    def init_grp(g, _):
        sl = pl.ds(g * 16, 16)
        dv = distv[sl]
        cl = clabv[sl]
        d016 = jnp.where(dv == 0, 1.0, 0.0)
        cm16 = jnp.where(cl != -1, 1.0, 0.0)
        d0v[sl] = d016
        cmv[sl] = cm16
        s0v[sl] = d016 * cm16
        return 0

    lax.fori_loop(0, MC // 16, init_grp, 0)

    def zero_c(c, _):
        sl = pl.ds(c * 16, 16)
        s1v[sl] = zv
        s2v[sl] = zv
        return 0

    lax.fori_loop(0, MC, zero_c, 0)

    def do_hop(first, dst):
        def hop_grp(grp, _):
            base = grp * 16
            gsl = pl.ds(base, 16)
            hv = headv[gsl]
            tv = tailv[gsl]
            gv = tmgv[gsl]
            for k in range(16):
                if first:
                    ns = plsc.load_gather(s0v, [_full16i(hv[k])])
                else:
                    ns = s1v[pl.ds(hv[k] * 16, 16)]
                upd = ns * gv[k] + tpv[pl.ds((base + k) * 16, 16)]
                sl = pl.ds(tv[k] * 16, 16)
                dst[sl] = jnp.maximum(dst[sl], upd)
            return 0

        lax.fori_loop(0, MT // 16, hop_grp, 0)

    do_hop(True, s1v)

    def mask_c(c, _):
        sl = pl.ds(c * 16, 16)
        s1v[sl] = s1v[sl] * plsc.load_gather(cmv, [_full16i(c)])
        return 0

    lax.fori_loop(0, MC, mask_c, 0)

    do_hop(False, s2v)

    itW = it * CPT_W

    def _tot_body(c, m):
        sl = pl.ds(c * 16, 16)
        d016 = plsc.load_gather(d0v, [_full16i(c)])
        cm16 = plsc.load_gather(cmv, [_full16i(c)])
        tv = (d016 + s1v[sl]) + s2v[sl] * cm16
        s1v[sl] = tv
        return jnp.maximum(m, tv)

    mx = lax.fori_loop(0, MC, _tot_body, jnp.full((16,), -jnp.inf))

    def _exp_body(c, acc):
        sl = pl.ds(c * 16, 16)
        e = jnp.exp(s1v[sl] - mx)
        s2v[sl] = e
        return acc + e

    ssum = lax.fori_loop(0, MC, _exp_body, zv)

    plsc.store_scatter(cptf, [itW + MC], zv)

    def norm_c(c, _):
        e = s2v[pl.ds(c * 16, 16)] / ssum
        plsc.store_scatter(cptf, [itW + c], e)
        return 0

    lax.fori_loop(0, MC, norm_c, 0)

    gv16 = gatev[pl.ds(lg * 16, 16)]

    for l_loc in range(16):
        g = gv16[l_loc]
        om = 1.0 - g
        l_glob = lg * 16 + l_loc
        row_base = l_loc * CPT_W

        def chunk_body(j, _, g=g, om=om, l_glob=l_glob, row_base=row_base):
            pltpu.sync_copy(lm_hbm.at[b, l_glob, pl.ds(j * CH, CH)], lmst)

            @plsc.parallel_loop(0, CH // 16, unroll=16)
            def _vec_body(i):
                o16 = i * 16
                sl = pl.ds(o16, 16)
                mapv = mapmv[pl.ds(j * CH + o16, 16)]
                cv = plsc.load_gather(cptf, [mapv + row_base])
                pst[sl] = cv * g + lmst[sl] * om
                cst[sl] = cv

            pltpu.sync_copy(pst, probs_hbm.at[b, l_glob, pl.ds(j * CH, CH)])
            pltpu.sync_copy(cst, cpv_hbm.at[b, l_glob, pl.ds(j * CH, CH)])
            return 0

        lax.fori_loop(0, V // CH, chunk_body, 0)


def _sc_call(tpt, gate2, lm_probs, mapm, head_idx, tail_idx, tmg,
             distances, concept_labels):
    mesh = plsc.VectorSubcoreMesh(core_axis_name="c", subcore_axis_name="s")
    fn = pl.kernel(
        _sc_body,
        out_type=[
            jax.ShapeDtypeStruct((B, L, V), jnp.float32),
            jax.ShapeDtypeStruct((B, L, V), jnp.float32),
        ],
        mesh=mesh,
        scratch_types=[
            pltpu.VMEM((MT * 16,), jnp.float32),
            pltpu.VMEM((MC,), jnp.float32),
            pltpu.VMEM((MC,), jnp.float32),
            pltpu.VMEM((MC,), jnp.float32),
            pltpu.VMEM((MC * 16,), jnp.float32),
            pltpu.VMEM((MC * 16,), jnp.float32),
            pltpu.VMEM((MT,), jnp.int32),
            pltpu.VMEM((MT,), jnp.int32),
            pltpu.VMEM((MT,), jnp.float32),
            pltpu.VMEM((MC,), jnp.int32),
            pltpu.VMEM((MC,), jnp.int32),
            pltpu.VMEM((16 * CPT_W,), jnp.float32),
            pltpu.VMEM((L,), jnp.float32),
            pltpu.VMEM((V,), jnp.int32),
            pltpu.VMEM((CH,), jnp.float32),
            pltpu.VMEM((CH,), jnp.float32),
            pltpu.VMEM((CH,), jnp.float32),
        ],
        compiler_params=pltpu.CompilerParams(needs_layout_passes=False),
    )
    return fn(tpt, gate2, lm_probs, mapm, head_idx, tail_idx, tmg,
              distances, concept_labels)


def kernel(lm_hidden_states, lm_probs, triple_repr, W_triple, W_gate, b_gate,
           triple_labels, vocab_map, map_mask, distances, concept_labels,
           head_idx, tail_idx):
    tp, tpt, gate2, tmg = _tc_dense(triple_repr, W_triple, lm_hidden_states,
                                    triple_labels, W_gate, b_gate)
    mapm = _tc_mapm(vocab_map, map_mask)
    probs, cpv = _sc_call(tpt, gate2, lm_probs, mapm, head_idx,
                          tail_idx, tmg, distances, concept_labels)
    isc = _tc_isc(probs, lm_probs)
    return (probs, gate2.reshape(B, L, 1), cpv, tp, isc)


# speedup vs baseline: 1.2764x; 1.1286x over previous
import functools

import jax
import jax.numpy as jnp
from jax import lax
from jax.experimental import pallas as pl
from jax.experimental.pallas import tpu as pltpu
from jax.experimental.pallas import tpu_sc as plsc

EMBED = 1024
NUM_HOPS = 2
GAMMA = 0.8
B, L, V, MT, MC = 4, 128, 32000, 1024, 512

MTB = 128
NLG = L // 16
CH = 6400
VB = 3200
CPT_W = MC + 8


def _tc_dense_body(trep_ref, w_ref, hid_ref, tl_ref, wg_ref, bg_ref,
                   vm_ref, mm_ref,
                   tp_ref, tpt_ref, gate_ref, tmg_ref, mapm_ref, hw_ref):
    m = pl.program_id(1)
    hid = hid_ref[0]

    @pl.when(m == 0)
    def _():
        w = w_ref[...]
        hw_ref[...] = lax.dot_general(hid, w, (((1,), (0,)), ((), ())),
                                      preferred_element_type=jnp.float32)
        wg = wg_ref[0]
        gsum = jnp.sum(hid * wg[None, :], axis=-1) + bg_ref[0, 0]
        gate_ref[0, 0] = jax.nn.sigmoid(gsum)
        mapm_ref[0, 0] = jnp.where(mm_ref[0, 0] != 0, vm_ref[0, 0], MC)

    tr = trep_ref[0]
    logits = lax.dot_general(hw_ref[...], tr, (((1,), (1,)), ((), ())),
                             preferred_element_type=jnp.float32)
    tl = tl_ref[0, 0, :]
    tmask = (tl != -1).astype(jnp.float32)
    tp = jax.nn.sigmoid(logits) * tmask[None, :]
    tp_ref[0] = tp
    tpt_ref[0] = jnp.transpose(tp.reshape(NLG, 16, MTB),
                               (0, 2, 1)).reshape(NLG, MTB * 16)
    tmg_ref[0, 0] = tmask * GAMMA


def _tc_dense(triple_repr, w_triple, hidden, tlabels, w_gate, b_gate,
              vocab_map, map_mask):
    tl3 = tlabels.reshape(B, 1, MT)
    bg2 = b_gate.reshape(1, 1)
    vm3 = vocab_map.reshape(B, 1, V)
    mm3 = map_mask.reshape(B, 1, V)
    grid = (B, MT // MTB)
    tp, tpt, gate2, tmg, mapm = pl.pallas_call(
        _tc_dense_body,
        grid=grid,
        in_specs=[
            pl.BlockSpec((1, MTB, 3 * EMBED), lambda b, m: (b, m, 0)),
            pl.BlockSpec((EMBED, 3 * EMBED), lambda b, m: (0, 0)),
            pl.BlockSpec((1, L, EMBED), lambda b, m: (b, 0, 0)),
            pl.BlockSpec((1, 1, MTB), lambda b, m: (b, 0, m)),
            pl.BlockSpec((1, EMBED), lambda b, m: (0, 0)),
            pl.BlockSpec((1, 1), lambda b, m: (0, 0)),
            pl.BlockSpec((1, 1, V), lambda b, m: (b, 0, 0)),
            pl.BlockSpec((1, 1, V), lambda b, m: (b, 0, 0)),
        ],
        out_specs=[
            pl.BlockSpec((1, L, MTB), lambda b, m: (b, 0, m)),
            pl.BlockSpec((1, NLG, MTB * 16), lambda b, m: (b, 0, m)),
            pl.BlockSpec((1, 1, L), lambda b, m: (b, 0, 0)),
            pl.BlockSpec((1, 1, MTB), lambda b, m: (b, 0, m)),
            pl.BlockSpec((1, 1, V), lambda b, m: (b, 0, 0)),
        ],
        out_shape=[
            jax.ShapeDtypeStruct((B, L, MT), jnp.float32),
            jax.ShapeDtypeStruct((B, NLG, MT * 16), jnp.float32),
            jax.ShapeDtypeStruct((B, 1, L), jnp.float32),
            jax.ShapeDtypeStruct((B, 1, MT), jnp.float32),
            jax.ShapeDtypeStruct((B, 1, V), jnp.int32),
        ],
        scratch_shapes=[pltpu.VMEM((L, 3 * EMBED), jnp.float32)],
        compiler_params=pltpu.CompilerParams(
            dimension_semantics=("parallel", "arbitrary")),
    )(triple_repr, w_triple, hidden, tl3, w_gate, bg2, vm3, mm3)
    return tp, tpt, gate2, tmg, mapm.reshape(B, V)


def _tc_isc_body(p_ref, lm_ref, isc_ref, pv_ref, pi_ref, lv_ref, li_ref):
    m = pl.program_id(1)

    def upd(x_ref, bv_ref, bi_ref):
        x = x_ref[0]
        loc = jnp.max(x, axis=-1)
        idx = jnp.argmax(x, axis=-1).astype(jnp.int32) + m * VB
        better = loc > bv_ref[0, 0]
        bi_ref[0, 0] = jnp.where(better, idx, bi_ref[0, 0])
        bv_ref[0, 0] = jnp.where(better, loc, bv_ref[0, 0])

    @pl.when(m == 0)
    def _():
        pv_ref[0, 0] = jnp.full((L,), -1.0, jnp.float32)
        lv_ref[0, 0] = jnp.full((L,), -1.0, jnp.float32)
        pi_ref[0, 0] = jnp.zeros((L,), jnp.int32)
        li_ref[0, 0] = jnp.zeros((L,), jnp.int32)

    upd(p_ref, pv_ref, pi_ref)
    upd(lm_ref, lv_ref, li_ref)

    @pl.when(m == V // VB - 1)
    def _():
        isc_ref[0, 0] = (pi_ref[0, 0] != li_ref[0, 0]).astype(jnp.int32)


def _tc_isc(probs, lm_probs):
    outs = pl.pallas_call(
        _tc_isc_body,
        grid=(B, V // VB),
        in_specs=[pl.BlockSpec((1, L, VB), lambda b, m: (b, 0, m)),
                  pl.BlockSpec((1, L, VB), lambda b, m: (b, 0, m))],
        out_specs=[pl.BlockSpec((1, 1, L), lambda b, m: (b, 0, 0))] * 5,
        out_shape=[jax.ShapeDtypeStruct((B, 1, L), jnp.int32),
                   jax.ShapeDtypeStruct((B, 1, L), jnp.float32),
                   jax.ShapeDtypeStruct((B, 1, L), jnp.int32),
                   jax.ShapeDtypeStruct((B, 1, L), jnp.float32),
                   jax.ShapeDtypeStruct((B, 1, L), jnp.int32)],
        compiler_params=pltpu.CompilerParams(
            dimension_semantics=("parallel", "arbitrary")),
    )(probs, lm_probs)
    return outs[0].reshape(B, L)


def _iota16():
    return lax.iota(jnp.int32, 16)


def _full16i(x):
    return jnp.full((16,), x, jnp.int32)


def _sc_body(tpt_hbm, gate_hbm, lm_hbm, mapm_hbm, head_hbm, tail_hbm,
             tmg_hbm, dist_hbm, clab_hbm,
             probs_hbm, cpv_hbm,
             tpv, s0v, d0v, cmv, s1v, s2v, headv, tailv, tmgv, distv, clabv,
             cptf, gatev, mapmv, lmst, pst, cst):
    wid = lax.axis_index("s") * 2 + lax.axis_index("c")
    b = wid // NLG
    lg = wid % NLG
    it = _iota16()
    zv = jnp.zeros((16,), jnp.float32)

    pltpu.sync_copy(tpt_hbm.at[b, lg], tpv)
    pltpu.sync_copy(gate_hbm.at[b, 0], gatev)
    pltpu.sync_copy(head_hbm.at[b], headv)
    pltpu.sync_copy(tail_hbm.at[b], tailv)
    pltpu.sync_copy(tmg_hbm.at[b, 0], tmgv)
    pltpu.sync_copy(dist_hbm.at[b], distv)
    pltpu.sync_copy(clab_hbm.at[b], clabv)
    pltpu.sync_copy(mapm_hbm.at[b], mapmv)

    def init_grp(g, _):
        sl = pl.ds(g * 16, 16)
        dv = distv[sl]
        cl = clabv[sl]
        d016 = jnp.where(dv == 0, 1.0, 0.0)
        cm16 = jnp.where(cl != -1, 1.0, 0.0)
        d0v[sl] = d016
        cmv[sl] = cm16
        s0v[sl] = d016 * cm16
        return 0

    lax.fori_loop(0, MC // 16, init_grp, 0)

    def zero_c(c, _):
        sl = pl.ds(c * 16, 16)
        s1v[sl] = zv
        s2v[sl] = zv
        return 0

    lax.fori_loop(0, MC, zero_c, 0)

    def do_hop(first, dst):
        def hop_grp(grp, _):
            base = grp * 16
            gsl = pl.ds(base, 16)
            hv = headv[gsl]
            tv = tailv[gsl]
            gv = tmgv[gsl]
            for k in range(16):
                if first:
                    ns = plsc.load_gather(s0v, [_full16i(hv[k])])
                else:
                    ns = s1v[pl.ds(hv[k] * 16, 16)]
                upd = ns * gv[k] + tpv[pl.ds((base + k) * 16, 16)]
                sl = pl.ds(tv[k] * 16, 16)
                dst[sl] = jnp.maximum(dst[sl], upd)
            return 0

        lax.fori_loop(0, MT // 16, hop_grp, 0)

    do_hop(True, s1v)

    def mask_c(c, _):
        sl = pl.ds(c * 16, 16)
        s1v[sl] = s1v[sl] * plsc.load_gather(cmv, [_full16i(c)])
        return 0

    lax.fori_loop(0, MC, mask_c, 0)

    do_hop(False, s2v)

    itW = it * CPT_W

    def _tot_body(c, m):
        sl = pl.ds(c * 16, 16)
        d016 = plsc.load_gather(d0v, [_full16i(c)])
        cm16 = plsc.load_gather(cmv, [_full16i(c)])
        tv = (d016 + s1v[sl]) + s2v[sl] * cm16
        s1v[sl] = tv
        return jnp.maximum(m, tv)

    mx = lax.fori_loop(0, MC, _tot_body, jnp.full((16,), -jnp.inf))

    def _exp_body(c, acc):
        sl = pl.ds(c * 16, 16)
        e = jnp.exp(s1v[sl] - mx)
        s2v[sl] = e
        return acc + e

    ssum = lax.fori_loop(0, MC, _exp_body, zv)

    plsc.store_scatter(cptf, [itW + MC], zv)

    def norm_c(c, _):
        e = s2v[pl.ds(c * 16, 16)] / ssum
        plsc.store_scatter(cptf, [itW + c], e)
        return 0

    lax.fori_loop(0, MC, norm_c, 0)

    gv16 = gatev[pl.ds(lg * 16, 16)]

    for l_loc in range(16):
        g = gv16[l_loc]
        om = 1.0 - g
        l_glob = lg * 16 + l_loc
        row_base = l_loc * CPT_W

        def chunk_body(j, _, g=g, om=om, l_glob=l_glob, row_base=row_base):
            pltpu.sync_copy(lm_hbm.at[b, l_glob, pl.ds(j * CH, CH)], lmst)

            @plsc.parallel_loop(0, CH // 16, unroll=16)
            def _vec_body(i):
                o16 = i * 16
                sl = pl.ds(o16, 16)
                mapv = mapmv[pl.ds(j * CH + o16, 16)]
                cv = plsc.load_gather(cptf, [mapv + row_base])
                pst[sl] = cv * g + lmst[sl] * om
                cst[sl] = cv

            pltpu.sync_copy(pst, probs_hbm.at[b, l_glob, pl.ds(j * CH, CH)])
            pltpu.sync_copy(cst, cpv_hbm.at[b, l_glob, pl.ds(j * CH, CH)])
            return 0

        lax.fori_loop(0, V // CH, chunk_body, 0)


def _sc_call(tpt, gate2, lm_probs, mapm, head_idx, tail_idx, tmg,
             distances, concept_labels):
    mesh = plsc.VectorSubcoreMesh(core_axis_name="c", subcore_axis_name="s")
    fn = pl.kernel(
        _sc_body,
        out_type=[
            jax.ShapeDtypeStruct((B, L, V), jnp.float32),
            jax.ShapeDtypeStruct((B, L, V), jnp.float32),
        ],
        mesh=mesh,
        scratch_types=[
            pltpu.VMEM((MT * 16,), jnp.float32),
            pltpu.VMEM((MC,), jnp.float32),
            pltpu.VMEM((MC,), jnp.float32),
            pltpu.VMEM((MC,), jnp.float32),
            pltpu.VMEM((MC * 16,), jnp.float32),
            pltpu.VMEM((MC * 16,), jnp.float32),
            pltpu.VMEM((MT,), jnp.int32),
            pltpu.VMEM((MT,), jnp.int32),
            pltpu.VMEM((MT,), jnp.float32),
            pltpu.VMEM((MC,), jnp.int32),
            pltpu.VMEM((MC,), jnp.int32),
            pltpu.VMEM((16 * CPT_W,), jnp.float32),
            pltpu.VMEM((L,), jnp.float32),
            pltpu.VMEM((V,), jnp.int32),
            pltpu.VMEM((CH,), jnp.float32),
            pltpu.VMEM((CH,), jnp.float32),
            pltpu.VMEM((CH,), jnp.float32),
        ],
        compiler_params=pltpu.CompilerParams(needs_layout_passes=False),
    )
    return fn(tpt, gate2, lm_probs, mapm, head_idx, tail_idx, tmg,
              distances, concept_labels)


def kernel(lm_hidden_states, lm_probs, triple_repr, W_triple, W_gate, b_gate,
           triple_labels, vocab_map, map_mask, distances, concept_labels,
           head_idx, tail_idx):
    tp, tpt, gate2, tmg, mapm = _tc_dense(triple_repr, W_triple,
                                          lm_hidden_states, triple_labels,
                                          W_gate, b_gate, vocab_map, map_mask)
    probs, cpv = _sc_call(tpt, gate2, lm_probs, mapm, head_idx,
                          tail_idx, tmg, distances, concept_labels)
    isc = _tc_isc(probs, lm_probs)
    return (probs, gate2.reshape(B, L, 1), cpv, tp, isc)
